# baseline (speedup 1.0000x reference)
import functools

import jax
import jax.numpy as jnp
from jax import lax
from jax.experimental import pallas as pl
from jax.experimental.pallas import tpu as pltpu
from jax.experimental.pallas import tpu_sc as plsc

D_MODEL = 1024
NUM_EXPERT = 8
TOP_K = 2
N_TOKENS = 32768
NOISE_EPS = 0.01

BLK = 2048
NW = 32
CHUNK = N_TOKENS // NW
L = 16
NEG = -jnp.inf


def _ncdf(z):
    x = z * 0.7071067811865476
    a = jnp.abs(x)
    t = 1.0 / (1.0 + 0.3275911 * a)
    poly = t * (0.254829592 + t * (-0.284496736 + t * (1.421413741
                + t * (-1.453152027 + t * 1.061405429))))
    erf_a = 1.0 - poly * jnp.exp(-a * a)
    return 0.5 * (1.0 + jnp.where(x < 0, -erf_a, erf_a))


def _producer_kernel(inp_ref, w_ref, noise_ref, clean_ref, stddev_ref,
                     noisy_ref):
    logits16 = lax.dot_general(
        w_ref[...], inp_ref[...],
        dimension_numbers=(((0,), (1,)), ((), ())),
        preferred_element_type=jnp.float32)
    clean = logits16[:NUM_EXPERT, :]
    raw = logits16[NUM_EXPERT:, :]
    stddev = (jnp.maximum(raw, 0.0)
              + jnp.log(1.0 + jnp.exp(-jnp.abs(raw))) + NOISE_EPS)
    clean_ref[...] = clean
    stddev_ref[...] = stddev
    noisy_ref[...] = clean + noise_ref[...] * stddev


def _producer(inp, w_cat, noise_t):
    grid = N_TOKENS // BLK
    return pl.pallas_call(
        _producer_kernel,
        grid=(grid,),
        in_specs=[
            pl.BlockSpec((BLK, D_MODEL), lambda i: (i, 0)),
            pl.BlockSpec((D_MODEL, 2 * NUM_EXPERT), lambda i: (0, 0)),
            pl.BlockSpec((NUM_EXPERT, BLK), lambda i: (0, i)),
        ],
        out_specs=[
            pl.BlockSpec((NUM_EXPERT, BLK), lambda i: (0, i)),
            pl.BlockSpec((NUM_EXPERT, BLK), lambda i: (0, i)),
            pl.BlockSpec((NUM_EXPERT, BLK), lambda i: (0, i)),
        ],
        out_shape=[
            jax.ShapeDtypeStruct((NUM_EXPERT, N_TOKENS), jnp.float32),
            jax.ShapeDtypeStruct((NUM_EXPERT, N_TOKENS), jnp.float32),
            jax.ShapeDtypeStruct((NUM_EXPERT, N_TOKENS), jnp.float32),
        ],
        compiler_params=pltpu.CompilerParams(
            dimension_semantics=("arbitrary",)),
    )(inp, w_cat, noise_t)


@functools.partial(
    pl.kernel,
    out_type=[
        jax.ShapeDtypeStruct((TOP_K, N_TOKENS), jnp.int32),
        jax.ShapeDtypeStruct((TOP_K, N_TOKENS), jnp.float32),
        jax.ShapeDtypeStruct((NW, NUM_EXPERT, L), jnp.float32),
        jax.ShapeDtypeStruct((NW, NUM_EXPERT, L), jnp.float32),
    ],
    mesh=plsc.VectorSubcoreMesh(core_axis_name="c", subcore_axis_name="s"),
    scratch_types=[
        pltpu.VMEM((NUM_EXPERT, CHUNK), jnp.float32),
        pltpu.VMEM((NUM_EXPERT, CHUNK), jnp.float32),
        pltpu.VMEM((NUM_EXPERT, CHUNK), jnp.float32),
        pltpu.VMEM((TOP_K, CHUNK), jnp.int32),
        pltpu.VMEM((TOP_K, CHUNK), jnp.float32),
        pltpu.VMEM((NUM_EXPERT, L), jnp.float32),
        pltpu.VMEM((NUM_EXPERT, L), jnp.float32),
    ],
)
def _sc_route(clean_hbm, stddev_hbm, noisy_hbm,
              idx_hbm, gates_hbm, load_hbm, imp_hbm,
              clean_v, stddev_v, noisy_v, idxbuf, gatebuf, loadp_v, impp_v):
    wid = lax.axis_index("s") * 2 + lax.axis_index("c")
    base = wid * CHUNK
    pltpu.sync_copy(clean_hbm.at[:, pl.ds(base, CHUNK)], clean_v)
    pltpu.sync_copy(stddev_hbm.at[:, pl.ds(base, CHUNK)], stddev_v)
    pltpu.sync_copy(noisy_hbm.at[:, pl.ds(base, CHUNK)], noisy_v)

    lane = lax.iota(jnp.int32, L)
    zero = jnp.zeros((L,), jnp.float32)

    def body(g, carry):
        accs = list(carry)
        off = g * L
        cl = [clean_v[e, pl.ds(off, L)] for e in range(NUM_EXPERT)]
        sd = [stddev_v[e, pl.ds(off, L)] for e in range(NUM_EXPERT)]
        nz = [noisy_v[e, pl.ds(off, L)] for e in range(NUM_EXPERT)]

        m1 = nz[0]
        for e in range(1, NUM_EXPERT):
            m1 = jnp.maximum(m1, nz[e])
        i1 = jnp.full((L,), NUM_EXPERT - 1, jnp.int32)
        for e in range(NUM_EXPERT - 2, -1, -1):
            i1 = jnp.where(nz[e] == m1, e, i1)
        v2 = [jnp.where(i1 == e, NEG, nz[e]) for e in range(NUM_EXPERT)]
        m2 = v2[0]
        for e in range(1, NUM_EXPERT):
            m2 = jnp.maximum(m2, v2[e])
        i2 = jnp.full((L,), NUM_EXPERT - 1, jnp.int32)
        for e in range(NUM_EXPERT - 2, -1, -1):
            i2 = jnp.where(v2[e] == m2, e, i2)
        m3 = jnp.where(i2 == 0, NEG, v2[0])
        for e in range(1, NUM_EXPERT):
            m3 = jnp.maximum(m3, jnp.where(i2 == e, NEG, v2[e]))

        t = jnp.exp(m2 - m1)
        g1 = 1.0 / (1.0 + t)
        g2 = 1.0 - g1

        idxbuf[0, pl.ds(off, L)] = i1
        idxbuf[1, pl.ds(off, L)] = i2
        gatebuf[0, pl.ds(off, L)] = g1
        gatebuf[1, pl.ds(off, L)] = g2

        for e in range(NUM_EXPERT):
            thr = jnp.where(nz[e] > m3, m3, m2)
            accs[e] = accs[e] + _ncdf((cl[e] - thr) / sd[e])
            accs[NUM_EXPERT + e] = (accs[NUM_EXPERT + e]
                                    + jnp.where(i1 == e, g1, zero)
                                    + jnp.where(i2 == e, g2, zero))
        return tuple(accs)

    init = tuple(zero for _ in range(2 * NUM_EXPERT))
    accs = lax.fori_loop(0, CHUNK // L, body, init)

    for e in range(NUM_EXPERT):
        loadp_v[e, :] = accs[e]
        impp_v[e, :] = accs[NUM_EXPERT + e]

    pltpu.sync_copy(idxbuf, idx_hbm.at[:, pl.ds(base, CHUNK)])
    pltpu.sync_copy(gatebuf, gates_hbm.at[:, pl.ds(base, CHUNK)])
    pltpu.sync_copy(loadp_v, load_hbm.at[wid])
    pltpu.sync_copy(impp_v, imp_hbm.at[wid])


def _loss_kernel(load_ref, imp_ref, loss_ref):
    def cv_sq(x):
        mean = jnp.sum(x) / NUM_EXPERT
        var = jnp.sum((x - mean) ** 2) / (NUM_EXPERT - 1)
        return var / (mean * mean + 1e-10)

    load = jnp.sum(jnp.sum(load_ref[...], axis=0), axis=1)
    imp = jnp.sum(jnp.sum(imp_ref[...], axis=0), axis=1)
    loss_ref[...] = jnp.broadcast_to(cv_sq(imp) + cv_sq(load), (1, 1))


def _finisher(load_parts, imp_parts):
    return pl.pallas_call(
        _loss_kernel,
        out_shape=jax.ShapeDtypeStruct((1, 1), jnp.float32),
    )(load_parts, imp_parts)


@jax.jit
def kernel(inp, w_gate, w_noise, noise):
    w_cat = jnp.concatenate([w_gate, w_noise], axis=1)
    clean_t, stddev_t, noisy_t = _producer(inp, w_cat, noise.T)
    idx, gates, load_parts, imp_parts = _sc_route(clean_t, stddev_t, noisy_t)
    loss = _finisher(load_parts, imp_parts)
    return (idx.T.reshape(-1), gates.T.reshape(N_TOKENS, 1, TOP_K),
            loss[0, 0])
